# Initial kernel scaffold; baseline (speedup 1.0000x reference)
import jax
import jax.numpy as jnp
from jax.experimental import pallas as pl


def kernel(input_nodes, src0, dst0, src1, dst1, emb, Ws0, bs0, Wn0, bn0, Ws1, bs1, Wn1, bn1, W_out, b_out):
    raise NotImplementedError("write your pallas kernel here")


# speedup vs baseline: 3.4254x; 3.4254x over previous
import functools

import jax
import jax.numpy as jnp
from jax import lax
from jax.experimental import pallas as pl
from jax.experimental.pallas import tpu as pltpu
from jax.experimental.pallas import tpu_sc as plsc

NC, NS, L = 2, 16, 16

H = 128
ND0 = 65536
ND1 = 4096
E0 = 524288
E1 = 65536

RPP = 8192
NPASS = 4
WE0 = E0 // NS
FC = 64
CMAX = WE0 + FC
GROW = RPP
DEGB0 = RPP + 8
ACCR0 = DEGB0 + 64
DEGB1 = ND1
ACCR1 = DEGB1 + 32
_CP = pltpu.CompilerParams(needs_layout_passes=False)

_mesh = plsc.VectorSubcoreMesh(
    core_axis_name="c", subcore_axis_name="s", num_cores=NC, num_subcores=NS
)


def _zero_rows(buf, rows):
    per = buf.shape[1] // L

    def body(i, _):
        buf[i // per, pl.ds((i % per) * L, L)] = jnp.zeros((L,), jnp.float32)
        return 0

    lax.fori_loop(0, rows * per, body, 0)


@functools.partial(
    pl.kernel,
    out_type=(
        jax.ShapeDtypeStruct((ND0, H), jnp.float32),
        jax.ShapeDtypeStruct((ND0 // H, H), jnp.float32),
        jax.ShapeDtypeStruct((ND0, H), jnp.float32),
    ),
    mesh=_mesh,
    compiler_params=_CP,
    scratch_types=[
        pltpu.VMEM((CMAX,), jnp.int32),
        pltpu.VMEM((FC, H), jnp.float32),
        pltpu.VMEM((FC,), jnp.int32),
        pltpu.VMEM((FC,), jnp.int32),
        pltpu.VMEM((1024,), jnp.int32),
        pltpu.VMEM((1024,), jnp.int32),
        pltpu.VMEM((64, H), jnp.float32),
        pltpu.VMEM((8, H), jnp.float32),
        pltpu.VMEM((1, FC), jnp.int32),
        pltpu.VMEM((1, 64), jnp.int32),
        pltpu.VMEM_SHARED((ACCR0, H), jnp.float32),
        pltpu.SemaphoreType.DMA,
        pltpu.SemaphoreType.DMA,
    ],
)
def _sc1(input_nodes, src0, dst0, emb, summed, deg, hself,
         cpk, rowA, idxA, sstage, srcb, dstb, deg2d, zbuf, dstage, i64,
         acc, semT, semR):
    cid = lax.axis_index("c")
    sid = lax.axis_index("s")

    _zero_rows(zbuf, 8)
    for t in range(4):
        i64[0, pl.ds(t * L, L)] = lax.iota(jnp.int32, L) + (DEGB0 + t * L)

    w = cid * NS + sid
    hbase = w * (ND0 // (NC * NS))

    def phase_h(j, _):
        b = hbase + j * FC
        pltpu.sync_copy(input_nodes.at[pl.ds(b, FC)], idxA)
        pltpu.async_copy(emb.at[idxA], rowA, semR).wait()
        pltpu.sync_copy(rowA, hself.at[pl.ds(b, FC)])
        return 0

    lax.fori_loop(0, (ND0 // (NC * NS)) // FC, phase_h, 0)

    def one_pass(p, _):
        rtar = p * NC + cid
        gbase = rtar * RPP

        def z1(i, _):
            pltpu.sync_copy(zbuf, acc.at[pl.ds(sid * 512 + i * 8, 8)])
            return 0

        lax.fori_loop(0, 512 // 8, z1, 0)

        @pl.when(sid == 0)
        def _():
            def zd(i, _):
                pltpu.sync_copy(zbuf, acc.at[pl.ds(DEGB0 + i * 8, 8)])
                return 0

            lax.fori_loop(0, 64 // 8, zd, 0)

        _zero_rows(deg2d, 64)
        plsc.subcore_barrier()

        def macro(m, n):
            eb = sid * WE0 + m * 1024
            pltpu.sync_copy(src0.at[pl.ds(eb, 1024)], srcb)
            pltpu.sync_copy(dst0.at[pl.ds(eb, 1024)], dstb)

            def step(k, n):
                s = srcb[pl.ds(k * L, L)]
                d = dstb[pl.ds(k * L, L)]
                msk = lax.shift_right_logical(d, 13) == rtar
                local = lax.bitwise_and(d, RPP - 1)
                plsc.addupdate_scatter(
                    deg2d,
                    [lax.shift_right_logical(local, 7),
                     lax.bitwise_and(local, H - 1)],
                    jnp.ones((L,), jnp.float32), mask=msk)
                ranks = plsc.cumsum(msk.astype(jnp.int32))
                pos = jnp.maximum(n + ranks - 1, 0)
                pk = lax.bitwise_or(lax.shift_left(local, 18), s)
                plsc.store_scatter(cpk, [pos], pk, mask=msk)
                return n + jnp.sum(msk.astype(jnp.int32))

            return lax.fori_loop(0, 1024 // L, step, n)

        n = lax.fori_loop(0, WE0 // 1024, macro, jnp.int32(0))

        iot = lax.iota(jnp.int32, L)
        for t in range(FC // L):
            plsc.store_scatter(cpk, [n + t * L + iot],
                               jnp.full((L,), GROW << 18, jnp.int32))

        nch = (n + FC - 1) // FC

        def flush(i, _):
            off = i * FC
            for t in range(FC // L):
                pk = cpk[pl.ds(off + t * L, L)]
                sstage[pl.ds(t * L, L)] = lax.bitwise_and(pk, (1 << 18) - 1)
                dstage[0, pl.ds(t * L, L)] = lax.shift_right_logical(pk, 18)
            pltpu.async_copy(input_nodes.at[sstage], idxA, semT).wait()
            pltpu.async_copy(emb.at[idxA], rowA, semR).wait()
            pltpu.sync_copy(rowA, acc.at[dstage.at[0]], add=True)
            return 0

        lax.fori_loop(0, nch, flush, 0)

        pltpu.sync_copy(deg2d, acc.at[i64.at[0]], add=True)
        plsc.subcore_barrier()

        pltpu.sync_copy(acc.at[pl.ds(sid * 512, 512)],
                        summed.at[pl.ds(gbase + sid * 512, 512)])

        @pl.when(sid == 0)
        def _():
            pltpu.sync_copy(acc.at[pl.ds(DEGB0, 64)],
                            deg.at[pl.ds(rtar * 64, 64)])

        plsc.subcore_barrier()
        return 0

    lax.fori_loop(0, NPASS, one_pass, 0)


@functools.partial(
    pl.kernel,
    out_type=(
        jax.ShapeDtypeStruct((NC, ND1, H), jnp.float32),
        jax.ShapeDtypeStruct((NC, ND1 // H, H), jnp.float32),
    ),
    mesh=_mesh,
    compiler_params=_CP,
    scratch_types=[
        pltpu.VMEM((2048,), jnp.int32),
        pltpu.VMEM((2048,), jnp.int32),
        pltpu.VMEM((128, H), jnp.float32),
        pltpu.VMEM((32, H), jnp.float32),
        pltpu.VMEM((8, H), jnp.float32),
        pltpu.VMEM((1, 128), jnp.int32),
        pltpu.VMEM((1, 32), jnp.int32),
        pltpu.VMEM_SHARED((ACCR1, H), jnp.float32),
        pltpu.SemaphoreType.DMA,
    ],
)
def _sc2(h0, src1, dst1, summed1, deg1,
         srcb, dstb, rowA, deg2d, zbuf, dstage, i32r, acc, semR):
    cid = lax.axis_index("c")
    sid = lax.axis_index("s")

    _zero_rows(zbuf, 8)
    _zero_rows(deg2d, 32)
    for t in range(2):
        i32r[0, pl.ds(t * L, L)] = lax.iota(jnp.int32, L) + (DEGB1 + t * L)

    def z1(i, _):
        pltpu.sync_copy(zbuf, acc.at[pl.ds(sid * 256 + i * 8, 8)])
        return 0

    lax.fori_loop(0, 256 // 8, z1, 0)

    @pl.when(sid == 0)
    def _():
        def zd(i, _):
            pltpu.sync_copy(zbuf, acc.at[pl.ds(DEGB1 + i * 8, 8)])
            return 0

        lax.fori_loop(0, 32 // 8, zd, 0)

    plsc.subcore_barrier()

    w = cid * NS + sid
    eb = w * (E1 // (NC * NS))
    pltpu.sync_copy(src1.at[pl.ds(eb, 2048)], srcb)
    pltpu.sync_copy(dst1.at[pl.ds(eb, 2048)], dstb)

    def degcount(k, _):
        d = dstb[pl.ds(k * L, L)]
        plsc.addupdate_scatter(
            deg2d,
            [lax.shift_right_logical(d, 7), lax.bitwise_and(d, H - 1)],
            jnp.ones((L,), jnp.float32))
        return 0

    lax.fori_loop(0, 2048 // L, degcount, 0)

    def chunk(k, _):
        off = k * 128
        pltpu.async_copy(h0.at[srcb.at[pl.ds(off, 128)]], rowA, semR).wait()
        for t in range(8):
            dstage[0, pl.ds(t * L, L)] = dstb[pl.ds(off + t * L, L)]
        pltpu.sync_copy(rowA, acc.at[dstage.at[0]], add=True)
        return 0

    lax.fori_loop(0, 2048 // 128, chunk, 0)
    pltpu.sync_copy(deg2d, acc.at[i32r.at[0]], add=True)
    plsc.subcore_barrier()

    pltpu.sync_copy(acc.at[pl.ds(sid * 256, 256)],
                    summed1.at[cid, pl.ds(sid * 256, 256)])

    @pl.when(sid == 0)
    def _():
        pltpu.sync_copy(acc.at[pl.ds(DEGB1, 32)], deg1.at[cid])


def _tc1_body(s_ref, d_ref, hs_ref, ws_ref, wn_ref, bs_ref, bn_ref, o_ref):
    neigh = s_ref[...] / jnp.maximum(d_ref[...], 1.0)
    dn = (((1,), (1,)), ((), ()))
    acc = lax.dot_general(hs_ref[...], ws_ref[...], dn,
                          preferred_element_type=jnp.float32)
    acc = acc + lax.dot_general(neigh, wn_ref[...], dn,
                                preferred_element_type=jnp.float32)
    o_ref[...] = jnp.maximum(acc + bs_ref[...] + bn_ref[...], 0.0)


def _tc2_body(hs_ref, s1_ref, d1_ref, ws_ref, wn_ref, wo_ref,
              bs_ref, bn_ref, bo_ref, o_ref):
    dg = d1_ref[0] + d1_ref[1]
    neigh = (s1_ref[0] + s1_ref[1]) / jnp.maximum(dg, 1.0)
    dn = (((1,), (1,)), ((), ()))
    h1 = lax.dot_general(hs_ref[...], ws_ref[...], dn,
                         preferred_element_type=jnp.float32)
    h1 = h1 + lax.dot_general(neigh, wn_ref[...], dn,
                              preferred_element_type=jnp.float32)
    h1 = h1 + bs_ref[...] + bn_ref[...]
    o_ref[...] = lax.dot_general(h1, wo_ref[...], dn,
                                 preferred_element_type=jnp.float32) + bo_ref[...]


_BLK = 2048


def _full(shape):
    return pl.BlockSpec(shape, lambda i: tuple(0 for _ in shape))


def kernel(input_nodes, src0, dst0, src1, dst1, emb,
           Ws0, bs0, Wn0, bn0, Ws1, bs1, Wn1, bn1, W_out, b_out):
    summed0, deg0, hself0 = _sc1(input_nodes, src0, dst0, emb)

    h0 = pl.pallas_call(
        _tc1_body,
        grid=(ND0 // _BLK,),
        in_specs=[
            pl.BlockSpec((_BLK, H), lambda i: (i, 0)),
            pl.BlockSpec((_BLK, 1), lambda i: (i, 0)),
            pl.BlockSpec((_BLK, H), lambda i: (i, 0)),
            _full((H, H)),
            _full((H, H)),
            _full((1, H)),
            _full((1, H)),
        ],
        out_specs=pl.BlockSpec((_BLK, H), lambda i: (i, 0)),
        out_shape=jax.ShapeDtypeStruct((ND0, H), jnp.float32),
    )(summed0, deg0.reshape(ND0, 1), hself0, Ws0, Wn0,
      bs0.reshape(1, H), bn0.reshape(1, H))

    summed1, deg1 = _sc2(h0, src1, dst1)

    logits = pl.pallas_call(
        _tc2_body,
        grid=(ND1 // _BLK,),
        in_specs=[
            pl.BlockSpec((_BLK, H), lambda i: (i, 0)),
            pl.BlockSpec((NC, _BLK, H), lambda i: (0, i, 0)),
            pl.BlockSpec((NC, _BLK, 1), lambda i: (0, i, 0)),
            _full((H, H)),
            _full((H, H)),
            _full((H, H)),
            _full((1, H)),
            _full((1, H)),
            _full((1, H)),
        ],
        out_specs=pl.BlockSpec((_BLK, H), lambda i: (i, 0)),
        out_shape=jax.ShapeDtypeStruct((ND1, H), jnp.float32),
    )(h0, summed1, deg1.reshape(NC, ND1, 1), Ws1, Wn1, W_out,
      bs1.reshape(1, H), bn1.reshape(1, H), b_out.reshape(1, H))

    return logits
